# baseline (speedup 1.0000x reference)
import jax
import jax.numpy as jnp
from jax.experimental import pallas as pl
from jax.experimental.pallas import tpu as pltpu

TOKEN_BLOCK = 512
NBUF = 4


def _router_mlp_kernel(x_hbm, w1_ref, b1_ref, w2_ref, b2_ref, o_ref,
                       x_bufs, sems):
    tokens = x_hbm.shape[0]
    n_blocks = tokens // TOKEN_BLOCK

    def copy_for(i, slot):
        return pltpu.make_async_copy(
            x_hbm.at[pl.ds(i * TOKEN_BLOCK, TOKEN_BLOCK), :],
            x_bufs.at[slot],
            sems.at[slot],
        )

    for j in range(NBUF):
        copy_for(j, j).start()

    def body(i, carry):
        slot = jax.lax.rem(i, NBUF)
        copy_for(i, slot).wait()
        h = jnp.dot(x_bufs[slot], w1_ref[...],
                    preferred_element_type=jnp.float32)
        h = jnp.maximum(h + b1_ref[...], 0.0)
        o_ref[pl.ds(i * TOKEN_BLOCK, TOKEN_BLOCK), :] = (
            jnp.dot(h, w2_ref[...], preferred_element_type=jnp.float32)
            + b2_ref[...]
        )
        nxt = i + NBUF

        @pl.when(nxt < n_blocks)
        def _():
            copy_for(nxt, slot).start()

        return carry

    jax.lax.fori_loop(0, n_blocks, body, 0)


@jax.jit
def kernel(x, W1, b1, W2, b2):
    tokens, input_dim = x.shape
    hidden = W1.shape[1]
    num_experts = W2.shape[1]
    b1 = b1.reshape(1, hidden)
    b2 = b2.reshape(1, num_experts)
    return pl.pallas_call(
        _router_mlp_kernel,
        in_specs=[
            pl.BlockSpec(memory_space=pl.ANY),
            pl.BlockSpec(memory_space=pltpu.VMEM),
            pl.BlockSpec(memory_space=pltpu.VMEM),
            pl.BlockSpec(memory_space=pltpu.VMEM),
            pl.BlockSpec(memory_space=pltpu.VMEM),
        ],
        out_specs=pl.BlockSpec(memory_space=pltpu.VMEM),
        out_shape=jax.ShapeDtypeStruct((tokens, num_experts), jnp.float32),
        scratch_shapes=[
            pltpu.VMEM((NBUF, TOKEN_BLOCK, input_dim), jnp.float32),
            pltpu.SemaphoreType.DMA((NBUF,)),
        ],
    )(x, W1, b1, W2, b2)
